# Initial kernel scaffold; baseline (speedup 1.0000x reference)
import jax
import jax.numpy as jnp
from jax.experimental import pallas as pl


def kernel(features, original_indices):
    raise NotImplementedError("write your pallas kernel here")


# speedup vs baseline: 6.0798x; 6.0798x over previous
import numpy as np
import jax
import jax.numpy as jnp
from jax.experimental import pallas as pl

_VOLUME = 8
_C = 16
_ROW = _VOLUME * _C


def _perm_matrix():
    P = np.zeros((_ROW, _ROW), dtype=np.float32)
    for i in range(_VOLUME):
        for c in range(_C):
            P[i * _C + c, c * _VOLUME + i] = 1.0
    return jnp.asarray(P)


def _body(x_ref, p_ref, o_ref):
    o_ref[...] = jax.lax.dot_general(
        x_ref[...], p_ref[...],
        dimension_numbers=(((1,), (0,)), ((), ())),
        precision=jax.lax.Precision.HIGHEST,
        preferred_element_type=jnp.float32,
    )


def kernel(features, original_indices):
    n_rows = features.shape[0] // _VOLUME
    x = features.reshape(n_rows, _ROW)
    P = _perm_matrix()
    bn = 2048
    out = pl.pallas_call(
        _body,
        grid=(n_rows // bn,),
        in_specs=[
            pl.BlockSpec((bn, _ROW), lambda i: (i, 0)),
            pl.BlockSpec((_ROW, _ROW), lambda i: (0, 0)),
        ],
        out_specs=pl.BlockSpec((bn, _ROW), lambda i: (i, 0)),
        out_shape=jax.ShapeDtypeStruct((n_rows, _ROW), jnp.float32),
    )(x, P)
    return out, original_indices
